# Initial kernel scaffold; baseline (speedup 1.0000x reference)
import jax
import jax.numpy as jnp
from jax.experimental import pallas as pl


def kernel(x, Wg, W1, W3, W2):
    raise NotImplementedError("write your pallas kernel here")


# speedup vs baseline: 1.4361x; 1.4361x over previous
import functools

import jax
import jax.numpy as jnp
from jax.experimental import pallas as pl
from jax.experimental.pallas import tpu as pltpu

H = 768
E = 8
INTER = 2048
TN = 256
NI = INTER // TN


def _moe_body(x_ref, wg_ref, w1_ref, w3_ref, w2_ref, out_ref, rw_ref):
    e = pl.program_id(0)
    ni = pl.program_id(1)
    xf = x_ref[...]

    @pl.when((e == 0) & (ni == 0))
    def _init():
        logits = jnp.dot(xf, wg_ref[...], preferred_element_type=jnp.float32)
        colid = jax.lax.broadcasted_iota(jnp.int32, logits.shape, 1)
        m1 = jnp.max(logits, axis=1, keepdims=True)
        idx1 = jnp.min(jnp.where(logits == m1, colid, E), axis=1, keepdims=True)
        sel1 = colid == idx1
        l2 = jnp.where(sel1, -jnp.inf, logits)
        m2 = jnp.max(l2, axis=1, keepdims=True)
        idx2 = jnp.min(jnp.where(l2 == m2, colid, E), axis=1, keepdims=True)
        sel2 = colid == idx2
        t = jnp.exp(m2 - m1)
        w_top = 1.0 / (1.0 + t)
        w_sec = t / (1.0 + t)
        rw_ref[...] = jnp.where(sel1, w_top, jnp.where(sel2, w_sec, 0.0))
        out_ref[...] = jnp.zeros_like(out_ref)

    a = jnp.dot(xf, w1_ref[0], preferred_element_type=jnp.float32)
    b = jnp.dot(xf, w3_ref[0], preferred_element_type=jnp.float32)
    h = (a * jax.nn.sigmoid(a)) * b
    colid = jax.lax.broadcasted_iota(jnp.int32, rw_ref.shape, 1)
    w_col = jnp.sum(jnp.where(colid == e, rw_ref[...], 0.0), axis=1, keepdims=True)
    out_ref[...] += jnp.dot(h * w_col, w2_ref[0], preferred_element_type=jnp.float32)


@jax.jit
def _moe(xf, Wg, W1, W3, W2):
    T = xf.shape[0]
    return pl.pallas_call(
        _moe_body,
        grid=(E, NI),
        in_specs=[
            pl.BlockSpec((T, H), lambda e, ni: (0, 0)),
            pl.BlockSpec((H, E), lambda e, ni: (0, 0)),
            pl.BlockSpec((1, H, TN), lambda e, ni: (e, 0, ni)),
            pl.BlockSpec((1, H, TN), lambda e, ni: (e, 0, ni)),
            pl.BlockSpec((1, TN, H), lambda e, ni: (e, ni, 0)),
        ],
        out_specs=pl.BlockSpec((T, H), lambda e, ni: (0, 0)),
        out_shape=jax.ShapeDtypeStruct((T, H), jnp.float32),
        scratch_shapes=[pltpu.VMEM((T, E), jnp.float32)],
        compiler_params=pltpu.CompilerParams(
            dimension_semantics=("arbitrary", "arbitrary"),
        ),
    )(xf, Wg, W1, W3, W2)


def kernel(x, Wg, W1, W3, W2):
    B, S, Hd = x.shape
    xf = x.reshape(-1, Hd)
    out = _moe(xf, Wg, W1, W3, W2)
    return out.reshape(B, S, Hd)


# speedup vs baseline: 1.6095x; 1.1207x over previous
import functools

import jax
import jax.numpy as jnp
from jax.experimental import pallas as pl
from jax.experimental.pallas import tpu as pltpu

H = 768
E = 8
INTER = 2048
TN = 256
NI = INTER // TN


def _moe_body(x_ref, wg_ref, w1_ref, w3_ref, w2_ref, out_ref, rw_ref):
    e = pl.program_id(0)
    ni = pl.program_id(1)

    @pl.when((e == 0) & (ni == 0))
    def _init():
        xf = x_ref[...]
        logits = jnp.dot(xf, wg_ref[...], preferred_element_type=jnp.float32)
        colid = jax.lax.broadcasted_iota(jnp.int32, logits.shape, 1)
        m1 = jnp.max(logits, axis=1, keepdims=True)
        idx1 = jnp.min(jnp.where(logits == m1, colid, E), axis=1, keepdims=True)
        sel1 = colid == idx1
        l2 = jnp.where(sel1, -jnp.inf, logits)
        m2 = jnp.max(l2, axis=1, keepdims=True)
        idx2 = jnp.min(jnp.where(l2 == m2, colid, E), axis=1, keepdims=True)
        sel2 = colid == idx2
        t = jnp.exp(m2 - m1)
        w_top = 1.0 / (1.0 + t)
        w_sec = t / (1.0 + t)
        rw_ref[...] = jnp.where(sel1, w_top, jnp.where(sel2, w_sec, 0.0))
        out_ref[...] = jnp.zeros_like(out_ref)

    xb = x_ref[...]
    a = jnp.dot(xb, w1_ref[0], preferred_element_type=jnp.float32,
                precision=jax.lax.Precision.DEFAULT)
    b = jnp.dot(xb, w3_ref[0], preferred_element_type=jnp.float32,
                precision=jax.lax.Precision.DEFAULT)
    h = (a * jax.nn.sigmoid(a)) * b
    colid = jax.lax.broadcasted_iota(jnp.int32, rw_ref.shape, 1)
    w_col = jnp.sum(jnp.where(colid == e, rw_ref[...], 0.0), axis=1, keepdims=True)
    out_ref[...] += jnp.dot(
        h * w_col, w2_ref[0],
        preferred_element_type=jnp.float32,
        precision=jax.lax.Precision.DEFAULT)


@jax.jit
def _moe(xf, Wg, W1, W3, W2):
    T = xf.shape[0]
    return pl.pallas_call(
        _moe_body,
        grid=(E, NI),
        in_specs=[
            pl.BlockSpec((T, H), lambda e, ni: (0, 0)),
            pl.BlockSpec((H, E), lambda e, ni: (0, 0)),
            pl.BlockSpec((1, H, TN), lambda e, ni: (e, 0, ni)),
            pl.BlockSpec((1, H, TN), lambda e, ni: (e, 0, ni)),
            pl.BlockSpec((1, TN, H), lambda e, ni: (e, ni, 0)),
        ],
        out_specs=pl.BlockSpec((T, H), lambda e, ni: (0, 0)),
        out_shape=jax.ShapeDtypeStruct((T, H), jnp.float32),
        scratch_shapes=[pltpu.VMEM((T, E), jnp.float32)],
        compiler_params=pltpu.CompilerParams(
            dimension_semantics=("arbitrary", "arbitrary"),
        ),
    )(xf, Wg, W1, W3, W2)


def kernel(x, Wg, W1, W3, W2):
    B, S, Hd = x.shape
    xf = x.reshape(-1, Hd)
    out = _moe(xf, Wg, W1, W3, W2)
    return out.reshape(B, S, Hd)
